# baseline (speedup 1.0000x reference)
import jax
import jax.numpy as jnp
from jax.experimental import pallas as pl
from jax.experimental.pallas import tpu as pltpu

_DEPTH = 1000
_N = 16384
_W = 2 * _DEPTH
_NR = _N // 2
_BR = 1024


def _onehot_block(idx_ref, out_ref):
    q = idx_ref[...]
    t0 = q[:, 0:1]
    t1 = q[:, 1:2] + _DEPTH
    cols = jax.lax.broadcasted_iota(jnp.int32, (_BR, _W), 1)
    hit = (cols == t0) | (cols == t1)
    out_ref[...] = jnp.where(hit, jnp.float32(1.0), jnp.float32(0.0))


def kernel(inputs):
    idx2 = inputs.astype(jnp.int32).reshape(_NR, 2)
    grid = _NR // _BR
    out2 = pl.pallas_call(
        _onehot_block,
        grid=(grid,),
        in_specs=[pl.BlockSpec((_BR, 2), lambda i: (i, 0))],
        out_specs=pl.BlockSpec((_BR, _W), lambda i: (i, 0)),
        out_shape=jax.ShapeDtypeStruct((_NR, _W), jnp.float32),
        compiler_params=pltpu.CompilerParams(
            dimension_semantics=("arbitrary",),
        ),
    )(idx2)
    return out2.reshape(_N, _DEPTH)
